# baseline (speedup 1.0000x reference)
import functools

import jax
import jax.numpy as jnp
from jax.experimental import pallas as pl
from jax.experimental.pallas import tpu as pltpu

_N = 512
_NSP = 200
_INTERPRET = False
_BF = jnp.bfloat16


def _relu(x):
    return jnp.maximum(x, 0.0)


def _bf(x):
    return x.astype(_BF)


def _mm(a, b):
    return jax.lax.dot_general(a, b, (((1,), (0,)), ((), ())),
                               preferred_element_type=jnp.float32)


def _tmm(a, b):
    return jax.lax.dot_general(a, b, (((0,), (0,)), ((), ())),
                               preferred_element_type=jnp.float32)


def _mmt(a, b):
    return jax.lax.dot_general(a, b, (((1,), (1,)), ((), ())),
                               preferred_element_type=jnp.float32)


def _eye(n, dtype):
    r = jax.lax.broadcasted_iota(jnp.int32, (n, n), 0)
    c = jax.lax.broadcasted_iota(jnp.int32, (n, n), 1)
    return (r == c).astype(dtype)


def _net(x, adj, w):
    (L1W, L1b, L2W, L2b, L3W, L3b, Wl1, Wl2, Wl3, bl, C1, C2, C3) = w
    A = adj + _eye(_N, _BF)
    deg = _mm(A, jnp.ones((_N, 1), _BF))
    invd = 1.0 / jnp.maximum(deg, 1e-6)

    def layer(prev_bf, W, b):
        xw = _bf(_mm(prev_bf, W))
        return _bf(_relu(_mm(A, xw) * invd + b))

    H1 = layer(x, L1W, L1b)
    H2 = layer(H1, L2W, L2b)
    H3 = layer(H2, L3W, L3b)

    s = _mm(H1, Wl1) + _mm(H2, Wl2) + _mm(H3, Wl3) + bl
    s = _relu(s)
    m = jnp.max(s, axis=1, keepdims=True)
    e = jnp.exp(s - m)
    s = _bf(e / jnp.sum(e, axis=1, keepdims=True))

    t = _bf(_tmm(s, adj))
    adj_p = _mm(t, s)
    M = (_mm(_bf(_tmm(s, H1)), C1) +
         _mm(_bf(_tmm(s, H2)), C2) +
         _mm(_bf(_tmm(s, H3)), C3))
    return adj_p, M


def _ranks(p):
    n = _NSP
    pj = p.reshape(n, 1)
    gt = (pj > p).astype(jnp.float32)
    ioj = jax.lax.broadcasted_iota(jnp.int32, (n, n), 0)
    ioi = jax.lax.broadcasted_iota(jnp.int32, (n, n), 1)
    eq = ((pj == p) & (ioj < ioi)).astype(jnp.float32)
    rank = jnp.sum(gt + eq, axis=0, keepdims=True)
    kk = jax.lax.broadcasted_iota(jnp.int32, (n, n), 0).astype(jnp.float32)
    return (rank == kk).astype(_BF)


def _pagerank_many(adjs):
    n = _NSP
    d, c = 0.85, (1.0 - 0.85) / _NSP

    def prep(a):
        deg = jnp.sum(a, axis=1, keepdims=True)
        return a / jnp.maximum(deg, 1e-6)

    As = [prep(a) for a in adjs]
    ps = [jnp.full((1, n), 1.0 / n, dtype=jnp.float32) for _ in As]
    for _ in range(10):
        ps = [d * jnp.dot(p, A, preferred_element_type=jnp.float32) + c
              for p, A in zip(ps, As)]
    return [_ranks(p) for p in ps]


def _body(xv_ref, av_ref, xp_ref, ap_ref, *rest):
    out_ref = rest[-1]
    w = [r[...] for r in rest[:-1]]
    wv, wp = tuple(w[0:13]), tuple(w[13:26])
    (ab1, aW2, ab2, aW3, ab3,
     l1a, l1b, l1c, l1bias, l2W, l2bias) = w[26:]
    G = xv_ref.shape[0]

    nets = []
    for g in range(G):
        nets.append(_net(xv_ref[g], av_ref[g], wv))
        nets.append(_net(xp_ref[g], ap_ref[g], wp))

    perms = _pagerank_many([adj for adj, _ in nets])

    for g in range(G):
        (adj1, M1), (adj3, M3) = nets[2 * g], nets[2 * g + 1]
        P1, P3 = perms[2 * g], perms[2 * g + 1]

        xW1 = _mm(P1, _bf(M1)) + _mm(P3, _bf(M3))
        adj1p = _mmt(_bf(_mm(P1, _bf(adj1))), P1)
        adj3p = _mmt(_bf(_mm(P3, _bf(adj3))), P3)

        A2 = _bf(adj1p + adj3p + _eye(_NSP, jnp.float32))
        deg2 = _mm(A2, jnp.ones((_NSP, 1), _BF))
        invd2 = 1.0 / jnp.maximum(deg2, 1e-6)

        a1 = _bf(_relu(_mm(A2, _bf(xW1)) * invd2 + ab1))
        a2 = _bf(_relu(_mm(A2, _bf(_mm(a1, aW2))) * invd2 + ab2))
        a3 = _bf(_relu(_mm(A2, _bf(_mm(a2, aW3))) * invd2 + ab3))

        ones_row = jnp.ones((1, _NSP), _BF)
        inv_n = 1.0 / _NSP
        g1 = _bf(_mm(ones_row, a1) * inv_n)
        g2 = _bf(_mm(ones_row, a2) * inv_n)
        g3 = _bf(_mm(ones_row, a3) * inv_n)

        h = _bf(_relu(_mm(g1, l1a) + _mm(g2, l1b) + _mm(g3, l1c) + l1bias))
        logits = _mm(h, l2W) + l2bias
        m = jnp.max(logits, axis=1, keepdims=True)
        lse = jnp.log(jnp.sum(jnp.exp(logits - m), axis=1, keepdims=True))
        out_ref[g] = logits - m - lse


def _blockdiag(a, b):
    za = jnp.zeros((a.shape[0], b.shape[1]), jnp.float32)
    zb = jnp.zeros((b.shape[0], a.shape[1]), jnp.float32)
    return jnp.concatenate([jnp.concatenate([a, za], axis=1),
                            jnp.concatenate([zb, b], axis=1)], axis=0)


def _net_weights(p, aW1_half):
    pool, emb = p['pool'], p['embed']
    H = 100
    Wl = pool['Wl']
    z100 = jnp.zeros((H, _NSP), jnp.float32)
    z100_400 = jnp.zeros((H, 400), jnp.float32)
    z200_400 = jnp.zeros((2 * H, 400), jnp.float32)
    return [
        _bf(jnp.concatenate([pool['W1'], emb['W1']], axis=1)),
        jnp.concatenate([pool['b1'], emb['b1']]).reshape(1, -1),
        _bf(_blockdiag(pool['W2'], emb['W2'])),
        jnp.concatenate([pool['b2'], emb['b2']]).reshape(1, -1),
        _bf(_blockdiag(pool['W3'], emb['W3'])),
        jnp.concatenate([pool['b3'], emb['b3']]).reshape(1, -1),
        _bf(jnp.concatenate([Wl[0:H], z100], axis=0)),
        _bf(jnp.concatenate([Wl[H:2 * H], z100], axis=0)),
        _bf(jnp.concatenate([Wl[2 * H:4 * H], z100], axis=0)),
        pool['bl'].reshape(1, -1),
        _bf(jnp.concatenate([z100_400, aW1_half[0:H]], axis=0)),
        _bf(jnp.concatenate([z100_400, aW1_half[H:2 * H]], axis=0)),
        _bf(jnp.concatenate([z200_400, aW1_half[2 * H:3 * H]], axis=0)),
    ]


def kernel(x_FV, adj_FV, x_FP, adj_FP, params):
    B = x_FV.shape[0]
    af = params['after']
    aW1 = af['W1']
    l1W = params['lin1_W']
    weights = (_net_weights(params['net_FV'], aW1[0:300]) +
               _net_weights(params['net_FP'], aW1[300:600]) + [
        af['b1'].reshape(1, -1), _bf(af['W2']), af['b2'].reshape(1, -1),
        _bf(af['W3']), af['b3'].reshape(1, -1),
        _bf(l1W[0:400]), _bf(l1W[400:800]), _bf(l1W[800:1200]),
        params['lin1_b'].reshape(1, -1),
        _bf(params['lin2_W']), params['lin2_b'].reshape(1, -1)])

    xv, xp = _bf(x_FV), _bf(x_FP)
    av, ap = _bf(adj_FV), _bf(adj_FP)
    D = xv.shape[2]

    G = 2
    data_specs = [
        pl.BlockSpec((G, _N, D), lambda b: (b, 0, 0)),
        pl.BlockSpec((G, _N, _N), lambda b: (b, 0, 0)),
        pl.BlockSpec((G, _N, D), lambda b: (b, 0, 0)),
        pl.BlockSpec((G, _N, _N), lambda b: (b, 0, 0)),
    ]
    w_specs = [pl.BlockSpec(w.shape, functools.partial(
        lambda nd, b: (0,) * nd, w.ndim)) for w in weights]

    out = pl.pallas_call(
        _body,
        grid=(B // G,),
        in_specs=data_specs + w_specs,
        out_specs=pl.BlockSpec((G, 1, 585), lambda b: (b, 0, 0)),
        out_shape=jax.ShapeDtypeStruct((B, 1, 585), jnp.float32),
        compiler_params=pltpu.CompilerParams(
            dimension_semantics=("parallel",)),
        interpret=_INTERPRET,
    )(xv, av, xp, ap, *weights)
    return out.reshape(B, 585)


# speedup vs baseline: 1.1677x; 1.1677x over previous
import functools

import jax
import jax.numpy as jnp
from jax.experimental import pallas as pl
from jax.experimental.pallas import tpu as pltpu

_N = 512
_NSP = 200
_INTERPRET = False
_BF = jnp.bfloat16


def _relu(x):
    return jnp.maximum(x, 0.0)


def _bf(x):
    return x.astype(_BF)


def _mm(a, b):
    return jax.lax.dot_general(a, b, (((1,), (0,)), ((), ())),
                               preferred_element_type=jnp.float32)


def _tmm(a, b):
    return jax.lax.dot_general(a, b, (((0,), (0,)), ((), ())),
                               preferred_element_type=jnp.float32)


def _mmt(a, b):
    return jax.lax.dot_general(a, b, (((1,), (1,)), ((), ())),
                               preferred_element_type=jnp.float32)


def _eye(n, dtype):
    r = jax.lax.broadcasted_iota(jnp.int32, (n, n), 0)
    c = jax.lax.broadcasted_iota(jnp.int32, (n, n), 1)
    return (r == c).astype(dtype)


def _net(x, adj_f32, w):
    (L1W, L1b, L2W, L2b, L3W, L3b, Wl1, Wl2, Wl3, bl, C1, C2, C3) = w
    adj = _bf(adj_f32)
    A = adj + _eye(_N, _BF)
    deg = _mm(A, jnp.ones((_N, 1), _BF))
    invd = 1.0 / jnp.maximum(deg, 1e-6)

    def layer(prev_bf, W, b):
        xw = _bf(_mm(prev_bf, W))
        return _bf(_relu(_mm(A, xw) * invd + b))

    H1 = layer(x, L1W, L1b)
    H2 = layer(H1, L2W, L2b)
    H3 = layer(H2, L3W, L3b)

    s = _mm(H1, Wl1) + _mm(H2, Wl2) + _mm(H3, Wl3) + bl
    s = _relu(s)
    m = jnp.max(s, axis=1, keepdims=True)
    e = jnp.exp(s - m)
    s = _bf(e / jnp.sum(e, axis=1, keepdims=True))

    t = _bf(_tmm(s, adj))
    adj_p = _mm(t, s)
    M = (_mm(_bf(_tmm(s, H1)), C1) +
         _mm(_bf(_tmm(s, H2)), C2) +
         _mm(_bf(_tmm(s, H3)), C3))
    return adj_p, M


def _ranks(p):
    n = _NSP
    pj = p.reshape(n, 1)
    gt = (pj > p).astype(jnp.float32)
    ioj = jax.lax.broadcasted_iota(jnp.int32, (n, n), 0)
    ioi = jax.lax.broadcasted_iota(jnp.int32, (n, n), 1)
    eq = ((pj == p) & (ioj < ioi)).astype(jnp.float32)
    rank = jnp.sum(gt + eq, axis=0, keepdims=True)
    kk = jax.lax.broadcasted_iota(jnp.int32, (n, n), 0).astype(jnp.float32)
    return (rank == kk).astype(_BF)


def _pagerank_many(adjs):
    n = _NSP
    d, c = 0.85, (1.0 - 0.85) / _NSP

    def prep(a):
        deg = jnp.sum(a, axis=1, keepdims=True)
        return a / jnp.maximum(deg, 1e-6)

    As = [prep(a) for a in adjs]
    ps = [jnp.full((1, n), 1.0 / n, dtype=jnp.float32) for _ in As]
    for _ in range(10):
        ps = [d * jnp.dot(p, A, preferred_element_type=jnp.float32) + c
              for p, A in zip(ps, As)]
    return [_ranks(p) for p in ps]


def _body(xv_ref, av_ref, xp_ref, ap_ref, *rest):
    out_ref = rest[-1]
    w = [r[...] for r in rest[:-1]]
    wv, wp = tuple(w[0:13]), tuple(w[13:26])
    (ab1, aW2, ab2, aW3, ab3,
     l1a, l1b, l1c, l1bias, l2W, l2bias) = w[26:]
    G = xv_ref.shape[0]

    nets = []
    for g in range(G):
        nets.append(_net(_bf(xv_ref[g]), av_ref[g], wv))
        nets.append(_net(_bf(xp_ref[g]), ap_ref[g], wp))

    perms = _pagerank_many([adj for adj, _ in nets])

    for g in range(G):
        (adj1, M1), (adj3, M3) = nets[2 * g], nets[2 * g + 1]
        P1, P3 = perms[2 * g], perms[2 * g + 1]

        xW1 = _mm(P1, _bf(M1)) + _mm(P3, _bf(M3))
        adj1p = _mmt(_bf(_mm(P1, _bf(adj1))), P1)
        adj3p = _mmt(_bf(_mm(P3, _bf(adj3))), P3)

        A2 = _bf(adj1p + adj3p + _eye(_NSP, jnp.float32))
        deg2 = _mm(A2, jnp.ones((_NSP, 1), _BF))
        invd2 = 1.0 / jnp.maximum(deg2, 1e-6)

        a1 = _bf(_relu(_mm(A2, _bf(xW1)) * invd2 + ab1))
        a2 = _bf(_relu(_mm(A2, _bf(_mm(a1, aW2))) * invd2 + ab2))
        a3 = _bf(_relu(_mm(A2, _bf(_mm(a2, aW3))) * invd2 + ab3))

        ones_row = jnp.ones((1, _NSP), _BF)
        inv_n = 1.0 / _NSP
        g1 = _bf(_mm(ones_row, a1) * inv_n)
        g2 = _bf(_mm(ones_row, a2) * inv_n)
        g3 = _bf(_mm(ones_row, a3) * inv_n)

        h = _bf(_relu(_mm(g1, l1a) + _mm(g2, l1b) + _mm(g3, l1c) + l1bias))
        logits = _mm(h, l2W) + l2bias
        m = jnp.max(logits, axis=1, keepdims=True)
        lse = jnp.log(jnp.sum(jnp.exp(logits - m), axis=1, keepdims=True))
        out_ref[g] = logits - m - lse


def _blockdiag(a, b):
    za = jnp.zeros((a.shape[0], b.shape[1]), jnp.float32)
    zb = jnp.zeros((b.shape[0], a.shape[1]), jnp.float32)
    return jnp.concatenate([jnp.concatenate([a, za], axis=1),
                            jnp.concatenate([zb, b], axis=1)], axis=0)


def _net_weights(p, aW1_half):
    pool, emb = p['pool'], p['embed']
    H = 100
    Wl = pool['Wl']
    z100 = jnp.zeros((H, _NSP), jnp.float32)
    z100_400 = jnp.zeros((H, 400), jnp.float32)
    z200_400 = jnp.zeros((2 * H, 400), jnp.float32)
    return [
        _bf(jnp.concatenate([pool['W1'], emb['W1']], axis=1)),
        jnp.concatenate([pool['b1'], emb['b1']]).reshape(1, -1),
        _bf(_blockdiag(pool['W2'], emb['W2'])),
        jnp.concatenate([pool['b2'], emb['b2']]).reshape(1, -1),
        _bf(_blockdiag(pool['W3'], emb['W3'])),
        jnp.concatenate([pool['b3'], emb['b3']]).reshape(1, -1),
        _bf(jnp.concatenate([Wl[0:H], z100], axis=0)),
        _bf(jnp.concatenate([Wl[H:2 * H], z100], axis=0)),
        _bf(jnp.concatenate([Wl[2 * H:4 * H], z100], axis=0)),
        pool['bl'].reshape(1, -1),
        _bf(jnp.concatenate([z100_400, aW1_half[0:H]], axis=0)),
        _bf(jnp.concatenate([z100_400, aW1_half[H:2 * H]], axis=0)),
        _bf(jnp.concatenate([z200_400, aW1_half[2 * H:3 * H]], axis=0)),
    ]


def kernel(x_FV, adj_FV, x_FP, adj_FP, params):
    B = x_FV.shape[0]
    af = params['after']
    aW1 = af['W1']
    l1W = params['lin1_W']
    weights = (_net_weights(params['net_FV'], aW1[0:300]) +
               _net_weights(params['net_FP'], aW1[300:600]) + [
        af['b1'].reshape(1, -1), _bf(af['W2']), af['b2'].reshape(1, -1),
        _bf(af['W3']), af['b3'].reshape(1, -1),
        _bf(l1W[0:400]), _bf(l1W[400:800]), _bf(l1W[800:1200]),
        params['lin1_b'].reshape(1, -1),
        _bf(params['lin2_W']), params['lin2_b'].reshape(1, -1)])

    xv, xp = x_FV, x_FP
    av, ap = adj_FV, adj_FP
    D = xv.shape[2]

    G = 2
    data_specs = [
        pl.BlockSpec((G, _N, D), lambda b: (b, 0, 0)),
        pl.BlockSpec((G, _N, _N), lambda b: (b, 0, 0)),
        pl.BlockSpec((G, _N, D), lambda b: (b, 0, 0)),
        pl.BlockSpec((G, _N, _N), lambda b: (b, 0, 0)),
    ]
    w_specs = [pl.BlockSpec(w.shape, functools.partial(
        lambda nd, b: (0,) * nd, w.ndim)) for w in weights]

    out = pl.pallas_call(
        _body,
        grid=(B // G,),
        in_specs=data_specs + w_specs,
        out_specs=pl.BlockSpec((G, 1, 585), lambda b: (b, 0, 0)),
        out_shape=jax.ShapeDtypeStruct((B, 1, 585), jnp.float32),
        compiler_params=pltpu.CompilerParams(
            dimension_semantics=("parallel",)),
        interpret=_INTERPRET,
    )(xv, av, xp, ap, *weights)
    return out.reshape(B, 585)
